# baseline (speedup 1.0000x reference)
import jax
import jax.numpy as jnp
from jax.experimental import pallas as pl

_BN = 20096


def _sims_body(f_ref, m_ref, o_ref):
    f = f_ref[0]
    norm = jnp.sqrt(jnp.sum(f * f, axis=1, keepdims=True))
    f = f / jnp.maximum(norm, 1e-12)
    m = m_ref[0]
    o_ref[0] = jax.lax.dot_general(
        f, m, (((1,), (1,)), ((), ())), preferred_element_type=jnp.float32
    )


def kernel(part_features, memory):
    k, b, d = part_features.shape
    _, n, _ = memory.shape
    bn = min(_BN, n)
    grid = (k, pl.cdiv(n, bn))
    return pl.pallas_call(
        _sims_body,
        grid=grid,
        in_specs=[
            pl.BlockSpec((1, b, d), lambda ki, ji: (ki, 0, 0)),
            pl.BlockSpec((1, bn, d), lambda ki, ji: (ki, ji, 0)),
        ],
        out_specs=pl.BlockSpec((1, b, bn), lambda ki, ji: (ki, 0, ji)),
        out_shape=jax.ShapeDtypeStruct((k, b, n), jnp.float32),
    )(part_features, memory)


# speedup vs baseline: 1.0053x; 1.0053x over previous
import jax
import jax.numpy as jnp
from jax.experimental import pallas as pl

_BN = 12544


def _sims_body(f_ref, m_ref, o_ref):
    f = f_ref[0]
    norm = jnp.sqrt(jnp.sum(f * f, axis=1, keepdims=True))
    f = f / jnp.maximum(norm, 1e-12)
    m = m_ref[0]
    o_ref[0] = jax.lax.dot_general(
        f, m, (((1,), (1,)), ((), ())), preferred_element_type=jnp.float32
    )


def kernel(part_features, memory):
    k, b, d = part_features.shape
    _, n, _ = memory.shape
    bn = min(_BN, n)
    grid = (k, pl.cdiv(n, bn))
    return pl.pallas_call(
        _sims_body,
        grid=grid,
        in_specs=[
            pl.BlockSpec((1, b, d), lambda ki, ji: (ki, 0, 0)),
            pl.BlockSpec((1, bn, d), lambda ki, ji: (ki, ji, 0)),
        ],
        out_specs=pl.BlockSpec((1, b, bn), lambda ki, ji: (ki, 0, ji)),
        out_shape=jax.ShapeDtypeStruct((k, b, n), jnp.float32),
    )(part_features, memory)
